# Initial kernel scaffold; baseline (speedup 1.0000x reference)
import jax
import jax.numpy as jnp
from jax.experimental import pallas as pl


def kernel(features, edge_index, D_norm, W0, b0, W1, b1):
    raise NotImplementedError("write your pallas kernel here")


# speedup vs baseline: 3.7920x; 3.7920x over previous
import functools

import jax
import jax.numpy as jnp
from jax import lax
from jax.experimental import pallas as pl
from jax.experimental.pallas import tpu as pltpu
from jax.experimental.pallas import tpu_sc as plsc

N = 10000
D = 128
E = 320000

NC = 2
NS = 16
CH = 128
GRP = 8
NCHUNK = ((-(-E // (NC * NS * CH)) + GRP - 1) // GRP) * GRP
EPAD = NC * NS * NCHUNK * CH
DUMMY = N
ZROWS = ((N + 1 + NS - 1) // NS + 7) // 8 * 8
N_ACC = ZROWS * NS


def _sc_segment_sum(features, src_r, dst_r):
    mesh = plsc.VectorSubcoreMesh(core_axis_name="c", subcore_axis_name="s")

    @functools.partial(
        pl.kernel,
        out_type=jax.ShapeDtypeStruct((NC, N_ACC, D), jnp.float32),
        mesh=mesh,
        scratch_types=[
            pltpu.VMEM((GRP, CH), jnp.int32),
            pltpu.VMEM((GRP, CH), jnp.int32),
            pltpu.VMEM((CH, D), jnp.float32),
            pltpu.VMEM_SHARED((N_ACC, D), jnp.float32),
            pltpu.SemaphoreType.DMA,
        ],
    )
    def seg_sum(feat_hbm, srci_hbm, dsti_hbm, part_hbm, srcv, dstv, rows, acc, sem):
        c = lax.axis_index("c")
        s = lax.axis_index("s")

        def zero_rows(i, carry):
            r = i // (D // 16)
            col = (i % (D // 16)) * 16
            rows[r, pl.ds(col, 16)] = jnp.zeros((16,), jnp.float32)
            return carry

        lax.fori_loop(0, CH * (D // 16), zero_rows, 0)

        base = s * ZROWS
        nfull = ZROWS // CH
        rem = ZROWS % CH

        def zero_acc(k, carry):
            pltpu.sync_copy(rows, acc.at[pl.ds(base + k * CH, CH)])
            return carry

        lax.fori_loop(0, nfull, zero_acc, 0)
        if rem:
            pltpu.sync_copy(rows.at[pl.ds(0, rem)],
                            acc.at[pl.ds(base + nfull * CH, rem)])
        plsc.subcore_barrier()

        def body(g, carry):
            pltpu.sync_copy(srci_hbm.at[c, s, pl.ds(g * GRP, GRP)], srcv)
            pltpu.sync_copy(dsti_hbm.at[c, s, pl.ds(g * GRP, GRP)], dstv)
            for b in range(GRP):
                pltpu.async_copy(feat_hbm.at[srcv.at[b]], rows, sem).wait()
                pltpu.sync_copy(rows, acc.at[dstv.at[b]], add=True)
            return carry

        lax.fori_loop(0, NCHUNK // GRP, body, 0)
        plsc.subcore_barrier()

        pltpu.sync_copy(acc.at[pl.ds(base, ZROWS)],
                        part_hbm.at[c, pl.ds(base, ZROWS)])

    return seg_sum(features, src_r, dst_r)


def _tc_tail(features, p0, p1, d_norm, W0, b0, W1, b1):
    R = 2000
    grid = (N // R,)

    def body(x_ref, p0_ref, p1_ref, d_ref, w0_ref, b0_ref, w1_ref, b1_ref, o_ref):
        x = x_ref[...]
        o_ref[:, :D] = (
            jnp.dot(x, w0_ref[...], preferred_element_type=jnp.float32)
            + b0_ref[...]
        )
        h = (p0_ref[...] + p1_ref[...]) * d_ref[...]
        o_ref[:, D:] = (
            jnp.dot(h, w1_ref[...], preferred_element_type=jnp.float32)
            + b1_ref[...]
        )

    return pl.pallas_call(
        body,
        grid=grid,
        in_specs=[
            pl.BlockSpec((R, D), lambda i: (i, 0)),
            pl.BlockSpec((R, D), lambda i: (i, 0)),
            pl.BlockSpec((R, D), lambda i: (i, 0)),
            pl.BlockSpec((R, 1), lambda i: (i, 0)),
            pl.BlockSpec((D, D), lambda i: (0, 0)),
            pl.BlockSpec((1, D), lambda i: (0, 0)),
            pl.BlockSpec((D, D), lambda i: (0, 0)),
            pl.BlockSpec((1, D), lambda i: (0, 0)),
        ],
        out_specs=pl.BlockSpec((R, 2 * D), lambda i: (i, 0)),
        out_shape=jax.ShapeDtypeStruct((N, 2 * D), jnp.float32),
    )(features, p0, p1, d_norm, W0, b0, W1, b1)


def kernel(features, edge_index, D_norm, W0, b0, W1, b1):
    src = edge_index[0].astype(jnp.int32)
    dst = edge_index[1].astype(jnp.int32)
    pad = EPAD - E
    src_r = jnp.concatenate([src, jnp.zeros((pad,), jnp.int32)])
    dst_r = jnp.concatenate([dst, jnp.full((pad,), DUMMY, jnp.int32)])
    src_r = src_r.reshape(NC, NS, NCHUNK, CH)
    dst_r = dst_r.reshape(NC, NS, NCHUNK, CH)

    part = _sc_segment_sum(features, src_r, dst_r)
    return _tc_tail(features, part[0], part[1], D_norm,
                    W0, b0.reshape(1, D), W1, b1.reshape(1, D))
